# baseline (speedup 1.0000x reference)
import functools

import jax
import jax.numpy as jnp
from jax import lax
from jax.experimental import pallas as pl
from jax.experimental.pallas import tpu as pltpu
from jax.experimental.pallas import tpu_sc as plsc

_N = 26
_VOCAB = 100000
_DIM = 16
_PAD = 32

_mesh = plsc.VectorSubcoreMesh(core_axis_name="c", subcore_axis_name="s")


@functools.partial(
    pl.kernel,
    mesh=_mesh,
    out_type=jax.ShapeDtypeStruct((_PAD, _DIM), jnp.float32),
    scratch_types=[
        pltpu.VMEM((_PAD,), jnp.int32),
        pltpu.VMEM((_PAD, _DIM), jnp.float32),
        pltpu.SemaphoreType.DMA,
    ],
    compiler_params=pltpu.CompilerParams(use_tc_tiling_on_sc=False),
)
def _gather(idx_hbm, table_hbm, out_hbm, idx_v, rows_v, sem):
    wid = lax.axis_index("s") * 2 + lax.axis_index("c")

    @pl.when(wid == 0)
    def _():
        pltpu.sync_copy(idx_hbm, idx_v)
        for j in range(_PAD // 16):
            fld = lax.iota(jnp.int32, 16) + (j * 16)
            fld = jnp.minimum(fld, _N - 1)
            idx_v[pl.ds(j * 16, 16)] = idx_v[pl.ds(j * 16, 16)] + fld * _VOCAB
        pltpu.async_copy(table_hbm.at[idx_v], rows_v, sem).wait()
        pltpu.sync_copy(rows_v, out_hbm)


def kernel(tables, indices):
    idx = jnp.pad(indices.astype(jnp.int32), (0, _PAD - _N))
    flat = tables.reshape(_N * _VOCAB, _DIM)
    out = _gather(idx, flat)
    return out[:_N].reshape(1, _N * _DIM)
